# Initial kernel scaffold; baseline (speedup 1.0000x reference)
import jax
import jax.numpy as jnp
from jax.experimental import pallas as pl


def kernel(buffer, spikes, delays, write_ptr):
    raise NotImplementedError("write your pallas kernel here")


# speedup vs baseline: 1.0633x; 1.0633x over previous
import functools

import jax
import jax.numpy as jnp
from jax import lax
from jax.experimental import pallas as pl
from jax.experimental.pallas import tpu as pltpu
from jax.experimental.pallas import tpu_sc as plsc

_D = 128
_N = 262144
_NC = 2
_NS = 16
_NW = _NC * _NS
_B = _N // _NW
_CW = 128
_CH = _B // _CW
_L = 16


def _sc_body(buf_hbm, spikes_hbm, delays_hbm, wp_hbm, out_hbm,
             delays_v, spikes_v, wp_v, idx_v, gath_v, sem):
    cid = lax.axis_index("c")
    sid = lax.axis_index("s")
    wid = sid * _NC + cid
    base = wid * _B

    pltpu.sync_copy(delays_hbm.at[wid], delays_v)
    pltpu.sync_copy(spikes_hbm.at[wid], spikes_v)
    pltpu.sync_copy(wp_hbm, wp_v)
    wp = wp_v[...]
    lane = lax.iota(jnp.int32, _L)

    def idx_body(c, carry):
        for s in range(_CW // _L):
            off = c * _CW + s * _L
            d = delays_v[pl.ds(off, _L)]
            r = jnp.bitwise_and(wp + (_D - d), _D - 1)
            idx_v[c, pl.ds(s * _L, _L)] = r * _N + (base + off + lane)
        return carry

    lax.fori_loop(0, _CH, idx_body, 0)

    copies = [pltpu.async_copy(buf_hbm.at[idx_v.at[c]], gath_v.at[c], sem)
              for c in range(_CH)]
    for cp in copies:
        cp.wait()

    def sel_body(c, carry):
        for s in range(_CW // _L):
            off = c * _CW + s * _L
            sl = pl.ds(s * _L, _L)
            d = delays_v[pl.ds(off, _L)]
            g = gath_v[c, sl]
            sp = spikes_v[pl.ds(off, _L)]
            gath_v[c, sl] = jnp.where(d == 0, sp, g)
        return carry

    lax.fori_loop(0, _CH, sel_body, 0)
    pltpu.sync_copy(gath_v, out_hbm.at[wid])


_sc_call = functools.partial(
    pl.kernel,
    out_type=jax.ShapeDtypeStruct((_NW, _CH, _CW), jnp.float32),
    mesh=plsc.VectorSubcoreMesh(core_axis_name="c", subcore_axis_name="s"),
    scratch_types=[
        pltpu.VMEM((_B,), jnp.int32),
        pltpu.VMEM((_B,), jnp.float32),
        pltpu.VMEM((_L,), jnp.int32),
        pltpu.VMEM((_CH, _CW), jnp.int32),
        pltpu.VMEM((_CH, _CW), jnp.float32),
        pltpu.SemaphoreType.DMA,
    ],
)(_sc_body)


def kernel(buffer, spikes, delays, write_ptr):
    buf_flat = buffer.reshape(_D * _N)
    wp_scalar = jnp.mod(jnp.asarray(write_ptr, jnp.int32), _D)
    wp = jnp.full((_L,), wp_scalar, jnp.int32)
    spikes2 = spikes.reshape(_NW, _B)
    delays2 = delays.astype(jnp.int32).reshape(_NW, _B)
    out = _sc_call(buf_flat, spikes2, delays2, wp)
    return out.reshape(_N)


# speedup vs baseline: 3.4428x; 3.2377x over previous
import functools

import jax
import jax.numpy as jnp
from jax import lax
from jax.experimental import pallas as pl
from jax.experimental.pallas import tpu as pltpu
from jax.experimental.pallas import tpu_sc as plsc

_D = 128
_N = 262144
_NC = 2
_NS = 16
_NW = _NC * _NS
_B = _N // _NW
_CW = 128
_CH = _B // _CW
_L = 16
_SUB = 8
_LANE = 128


def _sc_body(buf_hbm, spikes_hbm, delays_hbm, wp_hbm, out_hbm,
             delays_v, spikes_v, wp_v, idx_v, gath_v, sem):
    cid = lax.axis_index("c")
    sid = lax.axis_index("s")
    wid = sid * _NC + cid
    base = wid * _B

    pltpu.sync_copy(delays_hbm.at[wid], delays_v)
    pltpu.sync_copy(spikes_hbm.at[wid], spikes_v)
    pltpu.sync_copy(wp_hbm, wp_v)
    wp = wp_v[...]
    lane = lax.iota(jnp.int32, _L)

    def idx_body(c, carry):
        for s in range(_CW // _L):
            off = c * _CW + s * _L
            d = delays_v[pl.ds(off, _L)]
            r = jnp.bitwise_and(wp + (_D - d), _D - 1)
            col = base + off + lane
            idx_v[c, pl.ds(s * _L, _L)] = (
                (r >> 3) * (_SUB * (_N // _LANE) * _LANE)
                + (col >> 7) * (_SUB * _LANE)
                + jnp.bitwise_and(r, _SUB - 1) * _LANE
                + jnp.bitwise_and(col, _LANE - 1)
            )
        return carry

    lax.fori_loop(0, _CH, idx_body, 0)

    copies = [pltpu.async_copy(buf_hbm.at[idx_v.at[c]], gath_v.at[c], sem)
              for c in range(_CH)]
    for cp in copies:
        cp.wait()

    def sel_body(c, carry):
        for s in range(_CW // _L):
            off = c * _CW + s * _L
            sl = pl.ds(s * _L, _L)
            d = delays_v[pl.ds(off, _L)]
            g = gath_v[c, sl]
            sp = spikes_v[pl.ds(off, _L)]
            gath_v[c, sl] = jnp.where(d == 0, sp, g)
        return carry

    lax.fori_loop(0, _CH, sel_body, 0)
    pltpu.sync_copy(gath_v, out_hbm.at[wid])


_sc_call = functools.partial(
    pl.kernel,
    out_type=jax.ShapeDtypeStruct((_NW, _CH, _CW), jnp.float32),
    mesh=plsc.VectorSubcoreMesh(core_axis_name="c", subcore_axis_name="s"),
    scratch_types=[
        pltpu.VMEM((_B,), jnp.int32),
        pltpu.VMEM((_B,), jnp.float32),
        pltpu.VMEM((_L,), jnp.int32),
        pltpu.VMEM((_CH, _CW), jnp.int32),
        pltpu.VMEM((_CH, _CW), jnp.float32),
        pltpu.SemaphoreType.DMA,
    ],
)(_sc_body)


def kernel(buffer, spikes, delays, write_ptr):
    buf_flat = (buffer.reshape(_D // _SUB, _SUB, _N // _LANE, _LANE)
                .transpose(0, 2, 1, 3)
                .reshape(_D * _N))
    wp_scalar = jnp.mod(jnp.asarray(write_ptr, jnp.int32), _D)
    wp = jnp.full((_L,), wp_scalar, jnp.int32)
    spikes2 = spikes.reshape(_NW, _B)
    delays2 = delays.astype(jnp.int32).reshape(_NW, _B)
    out = _sc_call(buf_flat, spikes2, delays2, wp)
    return out.reshape(_N)


# speedup vs baseline: 3.8331x; 1.1134x over previous
import functools

import jax
import jax.numpy as jnp
from jax import lax
from jax.experimental import pallas as pl
from jax.experimental.pallas import tpu as pltpu
from jax.experimental.pallas import tpu_sc as plsc

_D = 128
_N = 262144
_NC = 2
_NS = 16
_NW = _NC * _NS
_B = _N // _NW
_CW = 128
_CH = _B // _CW
_L = 16
_SUB = 8
_LANE = 128


def _sc_body(buf_hbm, spikes_hbm, delays_hbm, wp_hbm, out_hbm,
             delays_v, spikes_v, wp_v, idx_v, gath_v, sem, spk_sem):
    cid = lax.axis_index("c")
    sid = lax.axis_index("s")
    wid = sid * _NC + cid
    base = wid * _B

    pltpu.sync_copy(delays_hbm.at[wid], delays_v)
    spk_cp = pltpu.async_copy(spikes_hbm.at[wid], spikes_v, spk_sem)
    pltpu.sync_copy(wp_hbm, wp_v)
    wp = wp_v[...]
    lane = lax.iota(jnp.int32, _L)

    def fire_body(c, carry):
        colpart0 = ((base >> 7) + c) * (_SUB * _LANE)
        for s in range(_CW // _L):
            off = c * _CW + s * _L
            d = delays_v[pl.ds(off, _L)]
            r = jnp.bitwise_and(wp + (_D - d), _D - 1)
            idx_v[c, pl.ds(s * _L, _L)] = (
                (r >> 3) * (_SUB * (_N // _LANE) * _LANE)
                + jnp.bitwise_and(r, _SUB - 1) * _LANE
                + (colpart0 + s * _L + lane)
            )
        pltpu.async_copy(buf_hbm.at[idx_v.at[c]], gath_v.at[c], sem)
        return carry

    lax.fori_loop(0, _CH, fire_body, 0)
    spk_cp.wait()

    def drain_body(c, carry):
        pltpu.make_async_copy(buf_hbm.at[idx_v.at[c]], gath_v.at[c], sem).wait()
        for s in range(_CW // _L):
            off = c * _CW + s * _L
            sl = pl.ds(s * _L, _L)
            d = delays_v[pl.ds(off, _L)]
            g = gath_v[c, sl]
            sp = spikes_v[pl.ds(off, _L)]
            gath_v[c, sl] = jnp.where(d == 0, sp, g)
        return carry

    lax.fori_loop(0, _CH, drain_body, 0)
    pltpu.sync_copy(gath_v, out_hbm.at[wid])


_sc_call = functools.partial(
    pl.kernel,
    out_type=jax.ShapeDtypeStruct((_NW, _CH, _CW), jnp.float32),
    mesh=plsc.VectorSubcoreMesh(core_axis_name="c", subcore_axis_name="s"),
    scratch_types=[
        pltpu.VMEM((_B,), jnp.int32),
        pltpu.VMEM((_B,), jnp.float32),
        pltpu.VMEM((_L,), jnp.int32),
        pltpu.VMEM((_CH, _CW), jnp.int32),
        pltpu.VMEM((_CH, _CW), jnp.float32),
        pltpu.SemaphoreType.DMA,
        pltpu.SemaphoreType.DMA,
    ],
)(_sc_body)


def kernel(buffer, spikes, delays, write_ptr):
    buf_flat = (buffer.reshape(_D // _SUB, _SUB, _N // _LANE, _LANE)
                .transpose(0, 2, 1, 3)
                .reshape(_D * _N))
    wp_scalar = jnp.mod(jnp.asarray(write_ptr, jnp.int32), _D)
    wp = jnp.full((_L,), wp_scalar, jnp.int32)
    spikes2 = spikes.reshape(_NW, _B)
    delays2 = delays.astype(jnp.int32).reshape(_NW, _B)
    out = _sc_call(buf_flat, spikes2, delays2, wp)
    return out.reshape(_N)
